# Initial kernel scaffold; baseline (speedup 1.0000x reference)
import jax
import jax.numpy as jnp
from jax.experimental import pallas as pl


def kernel(x, edge_index, params):
    raise NotImplementedError("write your pallas kernel here")


# speedup vs baseline: 9.3195x; 9.3195x over previous
import functools

import jax
import jax.numpy as jnp
from jax import lax
from jax.experimental import pallas as pl
from jax.experimental.pallas import tpu as pltpu
from jax.experimental.pallas import tpu_sc as plsc

_F32 = jnp.float32
_NC = 2
_NS = 16
_K = 80
_ZR = 125
_BR = 1000
_DEGW = 16


def _sc_mesh():
    return plsc.VectorSubcoreMesh(core_axis_name="c", subcore_axis_name="s",
                                  num_cores=_NC, num_subcores=_NS)


@functools.lru_cache(maxsize=None)
def _deg_kernel(N, E):
    EPC = E // _NC
    EPT = EPC // _NS
    nch = EPT // _K
    RPT = N // _NS

    def body(col_hbm, ones_hbm, zeros_hbm, out_hbm, colbuf, obuf, zbuf, acc):
        c = lax.axis_index("c")
        s = lax.axis_index("s")
        pltpu.sync_copy(zeros_hbm, zbuf)
        for z in range(RPT // _ZR):
            pltpu.sync_copy(zbuf, acc.at[pl.ds(s * RPT + z * _ZR, _ZR)])
        pltpu.sync_copy(ones_hbm, obuf)
        plsc.subcore_barrier()
        ebase = c * EPC + s * EPT

        def chunk(j, carry):
            pltpu.sync_copy(col_hbm.at[pl.ds(ebase + j * _K, _K)], colbuf)
            pltpu.sync_copy(obuf, acc.at[colbuf], add=True)
            return carry

        lax.fori_loop(0, nch, chunk, 0)
        plsc.subcore_barrier()
        pltpu.sync_copy(acc.at[pl.ds(s * RPT, RPT)],
                        out_hbm.at[c, pl.ds(s * RPT, RPT)])

    return pl.kernel(
        body,
        out_type=jax.ShapeDtypeStruct((_NC, N, _DEGW), _F32),
        mesh=_sc_mesh(),
        compiler_params=pltpu.CompilerParams(use_tc_tiling_on_sc=False),
        scratch_types=[
            pltpu.VMEM((_K,), jnp.int32),
            pltpu.VMEM((_K, _DEGW), _F32),
            pltpu.VMEM((_ZR, _DEGW), _F32),
            pltpu.VMEM_SHARED((N, _DEGW), _F32),
        ],
    )


@functools.lru_cache(maxsize=None)
def _agg_edge_split(N, E, do):
    EPC = E // _NC
    EPT = EPC // _NS
    nch = EPT // _K
    RPT = N // _NS

    def body(zt_hbm, row_hbm, col_hbm, zeros_hbm, out_hbm,
             rowbuf, colbuf, gbuf, zbuf, acc, sem):
        c = lax.axis_index("c")
        s = lax.axis_index("s")
        pltpu.sync_copy(zeros_hbm, zbuf)
        for z in range(RPT // _ZR):
            pltpu.sync_copy(zbuf, acc.at[pl.ds(s * RPT + z * _ZR, _ZR)])
        plsc.subcore_barrier()
        ebase = c * EPC + s * EPT

        def chunk(j, carry):
            eb = ebase + j * _K
            pltpu.sync_copy(row_hbm.at[pl.ds(eb, _K)], rowbuf)
            pltpu.sync_copy(col_hbm.at[pl.ds(eb, _K)], colbuf)
            pltpu.async_copy(zt_hbm.at[rowbuf], gbuf, sem).wait()
            pltpu.sync_copy(gbuf, acc.at[colbuf], add=True)
            return carry

        lax.fori_loop(0, nch, chunk, 0)
        plsc.subcore_barrier()
        pltpu.sync_copy(acc.at[pl.ds(s * RPT, RPT)],
                        out_hbm.at[c, pl.ds(s * RPT, RPT)])

    return pl.kernel(
        body,
        out_type=jax.ShapeDtypeStruct((_NC, N, do), _F32),
        mesh=_sc_mesh(),
        compiler_params=pltpu.CompilerParams(use_tc_tiling_on_sc=False),
        scratch_types=[
            pltpu.VMEM((_K,), jnp.int32),
            pltpu.VMEM((_K,), jnp.int32),
            pltpu.VMEM((_K, do), _F32),
            pltpu.VMEM((_ZR, do), _F32),
            pltpu.VMEM_SHARED((N, do), _F32),
            pltpu.SemaphoreType.DMA,
        ],
    )


@functools.lru_cache(maxsize=None)
def _agg_feat_split(N, E):
    DH = 128
    EPT = E // _NS
    nch = EPT // _K
    RPT = N // _NS

    def body(zt_hbm, row_hbm, col_hbm, zeros_hbm, out_hbm,
             rowbuf, colbuf, gbuf, zbuf, acc, sem):
        c = lax.axis_index("c")
        s = lax.axis_index("s")
        pltpu.sync_copy(zeros_hbm, zbuf)
        for z in range(RPT // _ZR):
            pltpu.sync_copy(zbuf, acc.at[pl.ds(s * RPT + z * _ZR, _ZR)])
        plsc.subcore_barrier()
        ebase = s * EPT

        def chunk(j, carry):
            eb = ebase + j * _K
            pltpu.sync_copy(row_hbm.at[pl.ds(eb, _K)], rowbuf)
            pltpu.sync_copy(col_hbm.at[pl.ds(eb, _K)], colbuf)
            for i in range(_K // 16):
                v = rowbuf[pl.ds(i * 16, 16)]
                rowbuf[pl.ds(i * 16, 16)] = v * 2 + c
            pltpu.async_copy(zt_hbm.at[rowbuf], gbuf, sem).wait()
            pltpu.sync_copy(gbuf, acc.at[colbuf], add=True)
            return carry

        lax.fori_loop(0, nch, chunk, 0)
        plsc.subcore_barrier()
        pltpu.sync_copy(acc.at[pl.ds(s * RPT, RPT)],
                        out_hbm.at[c, pl.ds(s * RPT, RPT)])

    return pl.kernel(
        body,
        out_type=jax.ShapeDtypeStruct((_NC, N, DH), _F32),
        mesh=_sc_mesh(),
        compiler_params=pltpu.CompilerParams(use_tc_tiling_on_sc=False),
        scratch_types=[
            pltpu.VMEM((_K,), jnp.int32),
            pltpu.VMEM((_K,), jnp.int32),
            pltpu.VMEM((_K, DH), _F32),
            pltpu.VMEM((_ZR, DH), _F32),
            pltpu.VMEM_SHARED((N, DH), _F32),
            pltpu.SemaphoreType.DMA,
        ],
    )


_SQRT1_2 = 0.7071067811865476


def _gelu(x):
    return x * 0.5 * (1.0 + lax.erf(x * _SQRT1_2))


def _full(shape):
    return pl.BlockSpec(shape, lambda i: tuple(0 for _ in shape))


def _dinv_tc(deg_parts):
    N = deg_parts.shape[1]

    def body(d_ref, o_ref):
        d = d_ref[0, :, 0:1] + d_ref[1, :, 0:1] + 1.0
        o_ref[...] = lax.rsqrt(d)

    return pl.pallas_call(
        body, out_shape=jax.ShapeDtypeStruct((N, 1), _F32))(deg_parts)


def _k_in(x, win, scale, shift, dinv, w0):
    N, DIN = x.shape
    H = win.shape[1]

    def body(x_ref, w_ref, s_ref, sh_ref, dv_ref, w0_ref, o_ref):
        u = jnp.dot(x_ref[...], w_ref[...], preferred_element_type=_F32)
        u = _gelu(u * s_ref[...] + sh_ref[...])
        o_ref[...] = dv_ref[...] * jnp.dot(
            u, w0_ref[...], preferred_element_type=_F32)

    return pl.pallas_call(
        body,
        grid=(N // _BR,),
        in_specs=[
            pl.BlockSpec((_BR, DIN), lambda i: (i, 0)),
            _full((DIN, H)),
            _full((1, H)),
            _full((1, H)),
            pl.BlockSpec((_BR, 1), lambda i: (i, 0)),
            _full((H, H)),
        ],
        out_specs=pl.BlockSpec((_BR, H), lambda i: (i, 0)),
        out_shape=jax.ShapeDtypeStruct((N, H), _F32),
    )(x, win, scale, shift, dinv, w0)


def _k_mid(parts, zt, dinv, scale, shift, lng, lnb, wnext, concat_mode):
    _, N, dh = parts.shape
    do = zt.shape[1]
    dn = wnext.shape[1]

    def body(p_ref, zt_ref, dv_ref, s_ref, sh_ref, g_ref, b_ref, w_ref, o_ref):
        if concat_mode:
            agg = jnp.concatenate([p_ref[0], p_ref[1]], axis=-1)
        else:
            agg = p_ref[0] + p_ref[1]
        z = dv_ref[...] * (agg + zt_ref[...])
        z = _gelu(z * s_ref[...] + sh_ref[...])
        m = jnp.mean(z, axis=-1, keepdims=True)
        zc = z - m
        v = jnp.mean(zc * zc, axis=-1, keepdims=True)
        z = zc * lax.rsqrt(v + 1e-5) * g_ref[...] + b_ref[...]
        o_ref[...] = dv_ref[...] * jnp.dot(
            z, w_ref[...], preferred_element_type=_F32)

    return pl.pallas_call(
        body,
        grid=(N // _BR,),
        in_specs=[
            pl.BlockSpec((2, _BR, dh), lambda i: (0, i, 0)),
            pl.BlockSpec((_BR, do), lambda i: (i, 0)),
            pl.BlockSpec((_BR, 1), lambda i: (i, 0)),
            _full((1, do)),
            _full((1, do)),
            _full((1, do)),
            _full((1, do)),
            _full((do, dn)),
        ],
        out_specs=pl.BlockSpec((_BR, dn), lambda i: (i, 0)),
        out_shape=jax.ShapeDtypeStruct((N, dn), _F32),
    )(parts, zt, dinv, scale, shift, lng, lnb, wnext)


def _k_last(parts, zt, dinv, scale, shift, lng, lnb,
            w1, b1, l1g, l1b, w2, b2, l2g, l2b, w3, b3, w4, b4):
    _, N, do = parts.shape
    d1 = w1.shape[1]
    d2 = w2.shape[1]
    d3 = w3.shape[1]

    def _ln(z, g, b):
        m = jnp.mean(z, axis=-1, keepdims=True)
        zc = z - m
        v = jnp.mean(zc * zc, axis=-1, keepdims=True)
        return zc * lax.rsqrt(v + 1e-5) * g + b

    def body(p_ref, zt_ref, dv_ref, s_ref, sh_ref, g_ref, b_ref,
             w1_ref, b1_ref, l1g_ref, l1b_ref, w2_ref, b2_ref,
             l2g_ref, l2b_ref, w3_ref, b3_ref, w4_ref, b4_ref, o_ref):
        z = dv_ref[...] * (p_ref[0] + p_ref[1] + zt_ref[...])
        z = _gelu(z * s_ref[...] + sh_ref[...])
        h = _ln(z, g_ref[...], b_ref[...])
        q = jnp.dot(h, w1_ref[...], preferred_element_type=_F32) + b1_ref[...]
        q = _gelu(_ln(q, l1g_ref[...], l1b_ref[...]))
        q = jnp.dot(q, w2_ref[...], preferred_element_type=_F32) + b2_ref[...]
        q = _gelu(_ln(q, l2g_ref[...], l2b_ref[...]))
        q = _gelu(jnp.dot(q, w3_ref[...], preferred_element_type=_F32)
                  + b3_ref[...])
        o_ref[...] = jnp.dot(q, w4_ref[...],
                             preferred_element_type=_F32) + b4_ref[...]

    return pl.pallas_call(
        body,
        grid=(N // _BR,),
        in_specs=[
            pl.BlockSpec((2, _BR, do), lambda i: (0, i, 0)),
            pl.BlockSpec((_BR, do), lambda i: (i, 0)),
            pl.BlockSpec((_BR, 1), lambda i: (i, 0)),
            _full((1, do)),
            _full((1, do)),
            _full((1, do)),
            _full((1, do)),
            _full((do, d1)),
            _full((1, d1)),
            _full((1, d1)),
            _full((1, d1)),
            _full((d1, d2)),
            _full((1, d2)),
            _full((1, d2)),
            _full((1, d2)),
            _full((d2, d3)),
            _full((1, d3)),
            _full((d3, 1)),
            _full((1, 1)),
        ],
        out_specs=pl.BlockSpec((_BR, 1), lambda i: (i, 0)),
        out_shape=jax.ShapeDtypeStruct((N, 1), _F32),
    )(parts, zt, dinv, scale, shift, lng, lnb,
      w1, b1, l1g, l1b, w2, b2, l2g, l2b, w3, b3, w4, b4)


def _bn_fold(bn, bias):
    s = bn["g"] * lax.rsqrt(bn["v"] + 1e-5)
    sh = (bias - bn["m"]) * s + bn["b"]
    return s[None, :], sh[None, :]


def _r2(v):
    return v[None, :]


def kernel(x, edge_index, params):
    p = params
    N, _ = x.shape
    E = edge_index.shape[1]
    row = edge_index[0]
    col = edge_index[1]

    deg_parts = _deg_kernel(N, E)(
        col, jnp.ones((_K, _DEGW), _F32), jnp.zeros((_ZR, _DEGW), _F32))
    dinv = _dinv_tc(deg_parts)

    layers = p["layers"]
    s_in, sh_in = _bn_fold(p["in_bn"], p["in_b"])
    zt = _k_in(x, p["in_W"], s_in, sh_in, dinv, layers[0]["W"])

    for i, layer in enumerate(layers):
        do = zt.shape[1]
        if do == 256:
            parts = _agg_feat_split(N, E)(
                zt.reshape(2 * N, 128), row, col, jnp.zeros((_ZR, 128), _F32))
        else:
            parts = _agg_edge_split(N, E, do)(
                zt, row, col, jnp.zeros((_ZR, do), _F32))
        s_i, sh_i = _bn_fold(layer["bn"], layer["b"])
        lng, lnb = _r2(layer["ln"]["g"]), _r2(layer["ln"]["b"])
        if i + 1 < len(layers):
            zt = _k_mid(parts, zt, dinv, s_i, sh_i, lng, lnb,
                        layers[i + 1]["W"], concat_mode=(do == 256))
        else:
            out = _k_last(
                parts, zt, dinv, s_i, sh_i, lng, lnb,
                p["p_W1"], _r2(p["p_b1"]), _r2(p["p_ln1g"]), _r2(p["p_ln1b"]),
                p["p_W2"], _r2(p["p_b2"]), _r2(p["p_ln2g"]), _r2(p["p_ln2b"]),
                p["p_W3"], _r2(p["p_b3"]), p["p_W4"], _r2(p["p_b4"]))
    return jnp.squeeze(out, -1)


# speedup vs baseline: 21.5528x; 2.3127x over previous
import functools

import jax
import jax.numpy as jnp
from jax import lax
from jax.experimental import pallas as pl
from jax.experimental.pallas import tpu as pltpu
from jax.experimental.pallas import tpu_sc as plsc

_F32 = jnp.float32
_NC = 2
_NS = 16
_K = 80
_BR = 1000


def _sc_mesh():
    return plsc.VectorSubcoreMesh(core_axis_name="c", subcore_axis_name="s",
                                  num_cores=_NC, num_subcores=_NS)


@functools.lru_cache(maxsize=None)
def _deg_kernel(N, E):
    EPT = E // (_NC * _NS)

    def body(col_hbm, out_hbm, colbuf, tacc):
        c = lax.axis_index("c")
        s = lax.axis_index("s")
        t = c * _NS + s
        pltpu.sync_copy(col_hbm.at[t], colbuf)
        zeros16 = jnp.zeros((16,), _F32)

        def zero(i, carry):
            tacc[pl.ds(i * 16, 16)] = zeros16
            return carry

        lax.fori_loop(0, N // 16, zero, 0)
        ones16 = jnp.full((16,), 1.0, _F32)

        def add(i, carry):
            idxv = colbuf[pl.ds(i * 16, 16)]
            plsc.addupdate_scatter(tacc, [idxv], ones16)
            return carry

        lax.fori_loop(0, EPT // 16, add, 0)
        pltpu.sync_copy(tacc, out_hbm.at[t])

    return pl.kernel(
        body,
        out_type=jax.ShapeDtypeStruct((_NC * _NS, N), _F32),
        mesh=_sc_mesh(),
        scratch_types=[
            pltpu.VMEM((EPT,), jnp.int32),
            pltpu.VMEM((N,), _F32),
        ],
        compiler_params=pltpu.CompilerParams(
            use_tc_tiling_on_sc=False, needs_layout_passes=False),
    )


def _agg_impl(nch, RPT, ZR, nbuf):
    def impl(zt_hbm, row_hbm, col_hbm, zeros_hbm, out_hbm,
             idxr, idxc, gbufs, zbuf, acc, gsems, ssems):
        c = lax.axis_index("c")
        s = lax.axis_index("s")
        t = c * _NS + s
        pltpu.sync_copy(row_hbm.at[t], idxr)
        pltpu.sync_copy(col_hbm.at[t], idxc)
        pltpu.sync_copy(zeros_hbm, zbuf)
        for z in range(RPT // ZR):
            pltpu.sync_copy(zbuf, acc.at[pl.ds(s * RPT + z * ZR, ZR)])
        plsc.subcore_barrier()

        def gather(j, b):
            pltpu.async_copy(zt_hbm.at[idxr.at[j]], gbufs[b], gsems[b])

        def scatter(j, b):
            pltpu.async_copy(gbufs[b], acc.at[idxc.at[j]], ssems[b],
                             add=True)

        def wait_g(b):
            pltpu.make_async_copy(
                zt_hbm.at[idxr.at[0]], gbufs[b], gsems[b]).wait()

        def wait_s(b):
            pltpu.make_async_copy(
                gbufs[b], acc.at[idxc.at[0]], ssems[b]).wait()

        nrounds, rem = divmod(nch, nbuf)
        for b in range(nbuf):
            gather(b, b)

        def round_body(r, carry):
            j0 = r * nbuf
            for b in range(nbuf):
                wait_g(b)
                scatter(j0 + b, b)
            for b in range(nbuf):
                wait_s(b)
                nj = j0 + nbuf + b

                @pl.when(nj < nch)
                def _():
                    gather(nj, b)
            return carry

        lax.fori_loop(0, nrounds, round_body, 0)
        for i in range(rem):
            wait_g(i)
            scatter(nrounds * nbuf + i, i)
        for i in range(rem):
            wait_s(i)
        plsc.subcore_barrier()
        pltpu.sync_copy(acc.at[pl.ds(s * RPT, RPT)],
                        out_hbm.at[c, pl.ds(s * RPT, RPT)])

    return impl


@functools.lru_cache(maxsize=None)
def _agg_edge_split(N, E, do):
    EPT = E // (_NC * _NS)
    nch = EPT // _K
    RPT = N // _NS
    nbuf = 2 if do >= 128 else 4
    ZR = 25 if do >= 128 else 125
    impl = _agg_impl(nch, RPT, ZR, nbuf)

    if nbuf == 2:
        def body(zt, rw, cl, zs, out, idxr, idxc, g0, g1, zb, acc,
                 gs0, gs1, ss0, ss1):
            impl(zt, rw, cl, zs, out, idxr, idxc, (g0, g1), zb, acc,
                 (gs0, gs1), (ss0, ss1))
    else:
        def body(zt, rw, cl, zs, out, idxr, idxc, g0, g1, g2, g3, zb, acc,
                 gs0, gs1, gs2, gs3, ss0, ss1, ss2, ss3):
            impl(zt, rw, cl, zs, out, idxr, idxc, (g0, g1, g2, g3), zb, acc,
                 (gs0, gs1, gs2, gs3), (ss0, ss1, ss2, ss3))

    scratch = (
        [pltpu.VMEM((nch, _K), jnp.int32)] * 2
        + [pltpu.VMEM((_K, do), _F32)] * nbuf
        + [pltpu.VMEM((ZR, do), _F32),
           pltpu.VMEM_SHARED((N, do), _F32)]
        + [pltpu.SemaphoreType.DMA] * (2 * nbuf)
    )
    return pl.kernel(
        body,
        out_type=jax.ShapeDtypeStruct((_NC, N, do), _F32),
        mesh=_sc_mesh(),
        scratch_types=scratch,
        compiler_params=pltpu.CompilerParams(use_tc_tiling_on_sc=False),
    )


_SQRT1_2 = 0.7071067811865476


def _gelu(x):
    return x * 0.5 * (1.0 + lax.erf(x * _SQRT1_2))


def _full(shape):
    return pl.BlockSpec(shape, lambda i: tuple(0 for _ in shape))


def _dinv_tc(deg_parts):
    N = deg_parts.shape[1]

    def body(d_ref, o_ref):
        d = jnp.sum(d_ref[...], axis=0) + 1.0
        o_ref[...] = lax.rsqrt(d)[:, None]

    return pl.pallas_call(
        body, out_shape=jax.ShapeDtypeStruct((N, 1), _F32))(deg_parts)


def _k_in(x, win, scale, shift, dinv, w0):
    N, DIN = x.shape
    H = win.shape[1]
    HH = H // 2

    def body(x_ref, w_ref, s_ref, sh_ref, dv_ref, w0_ref, oa_ref, ob_ref):
        u = jnp.dot(x_ref[...], w_ref[...], preferred_element_type=_F32)
        u = _gelu(u * s_ref[...] + sh_ref[...])
        zt = dv_ref[...] * jnp.dot(u, w0_ref[...], preferred_element_type=_F32)
        oa_ref[...] = zt[:, :HH]
        ob_ref[...] = zt[:, HH:]

    return pl.pallas_call(
        body,
        grid=(N // _BR,),
        in_specs=[
            pl.BlockSpec((_BR, DIN), lambda i: (i, 0)),
            _full((DIN, H)),
            _full((1, H)),
            _full((1, H)),
            pl.BlockSpec((_BR, 1), lambda i: (i, 0)),
            _full((H, H)),
        ],
        out_specs=[pl.BlockSpec((_BR, HH), lambda i: (i, 0)),
                   pl.BlockSpec((_BR, HH), lambda i: (i, 0))],
        out_shape=[jax.ShapeDtypeStruct((N, HH), _F32),
                   jax.ShapeDtypeStruct((N, HH), _F32)],
    )(x, win, scale, shift, dinv, w0)


def _k_mid0(partsa, partsb, zta, ztb, dinv, scale, shift, lng, lnb, wnext):
    _, N, dh = partsa.shape
    dn = wnext.shape[1]
    do = 2 * dh

    def body(pa_ref, pb_ref, za_ref, zb_ref, dv_ref, s_ref, sh_ref,
             g_ref, b_ref, w_ref, o_ref):
        agg = jnp.concatenate(
            [pa_ref[0] + pa_ref[1] + za_ref[...],
             pb_ref[0] + pb_ref[1] + zb_ref[...]], axis=-1)
        z = dv_ref[...] * agg
        z = _gelu(z * s_ref[...] + sh_ref[...])
        m = jnp.mean(z, axis=-1, keepdims=True)
        zc = z - m
        v = jnp.mean(zc * zc, axis=-1, keepdims=True)
        z = zc * lax.rsqrt(v + 1e-5) * g_ref[...] + b_ref[...]
        o_ref[...] = dv_ref[...] * jnp.dot(
            z, w_ref[...], preferred_element_type=_F32)

    return pl.pallas_call(
        body,
        grid=(N // _BR,),
        in_specs=[
            pl.BlockSpec((2, _BR, dh), lambda i: (0, i, 0)),
            pl.BlockSpec((2, _BR, dh), lambda i: (0, i, 0)),
            pl.BlockSpec((_BR, dh), lambda i: (i, 0)),
            pl.BlockSpec((_BR, dh), lambda i: (i, 0)),
            pl.BlockSpec((_BR, 1), lambda i: (i, 0)),
            _full((1, do)),
            _full((1, do)),
            _full((1, do)),
            _full((1, do)),
            _full((do, dn)),
        ],
        out_specs=pl.BlockSpec((_BR, dn), lambda i: (i, 0)),
        out_shape=jax.ShapeDtypeStruct((N, dn), _F32),
    )(partsa, partsb, zta, ztb, dinv, scale, shift, lng, lnb, wnext)


def _k_mid(parts, zt, dinv, scale, shift, lng, lnb, wnext):
    _, N, do = parts.shape
    dn = wnext.shape[1]

    def body(p_ref, zt_ref, dv_ref, s_ref, sh_ref, g_ref, b_ref, w_ref, o_ref):
        agg = p_ref[0] + p_ref[1]
        z = dv_ref[...] * (agg + zt_ref[...])
        z = _gelu(z * s_ref[...] + sh_ref[...])
        m = jnp.mean(z, axis=-1, keepdims=True)
        zc = z - m
        v = jnp.mean(zc * zc, axis=-1, keepdims=True)
        z = zc * lax.rsqrt(v + 1e-5) * g_ref[...] + b_ref[...]
        o_ref[...] = dv_ref[...] * jnp.dot(
            z, w_ref[...], preferred_element_type=_F32)

    return pl.pallas_call(
        body,
        grid=(N // _BR,),
        in_specs=[
            pl.BlockSpec((2, _BR, do), lambda i: (0, i, 0)),
            pl.BlockSpec((_BR, do), lambda i: (i, 0)),
            pl.BlockSpec((_BR, 1), lambda i: (i, 0)),
            _full((1, do)),
            _full((1, do)),
            _full((1, do)),
            _full((1, do)),
            _full((do, dn)),
        ],
        out_specs=pl.BlockSpec((_BR, dn), lambda i: (i, 0)),
        out_shape=jax.ShapeDtypeStruct((N, dn), _F32),
    )(parts, zt, dinv, scale, shift, lng, lnb, wnext)


def _k_last(parts, zt, dinv, scale, shift, lng, lnb,
            w1, b1, l1g, l1b, w2, b2, l2g, l2b, w3, b3, w4, b4):
    _, N, do = parts.shape
    d1 = w1.shape[1]
    d2 = w2.shape[1]
    d3 = w3.shape[1]

    def _ln(z, g, b):
        m = jnp.mean(z, axis=-1, keepdims=True)
        zc = z - m
        v = jnp.mean(zc * zc, axis=-1, keepdims=True)
        return zc * lax.rsqrt(v + 1e-5) * g + b

    def body(p_ref, zt_ref, dv_ref, s_ref, sh_ref, g_ref, b_ref,
             w1_ref, b1_ref, l1g_ref, l1b_ref, w2_ref, b2_ref,
             l2g_ref, l2b_ref, w3_ref, b3_ref, w4_ref, b4_ref, o_ref):
        z = dv_ref[...] * (p_ref[0] + p_ref[1] + zt_ref[...])
        z = _gelu(z * s_ref[...] + sh_ref[...])
        h = _ln(z, g_ref[...], b_ref[...])
        q = jnp.dot(h, w1_ref[...], preferred_element_type=_F32) + b1_ref[...]
        q = _gelu(_ln(q, l1g_ref[...], l1b_ref[...]))
        q = jnp.dot(q, w2_ref[...], preferred_element_type=_F32) + b2_ref[...]
        q = _gelu(_ln(q, l2g_ref[...], l2b_ref[...]))
        q = _gelu(jnp.dot(q, w3_ref[...], preferred_element_type=_F32)
                  + b3_ref[...])
        o_ref[...] = jnp.dot(q, w4_ref[...],
                             preferred_element_type=_F32) + b4_ref[...]

    return pl.pallas_call(
        body,
        grid=(N // _BR,),
        in_specs=[
            pl.BlockSpec((2, _BR, do), lambda i: (0, i, 0)),
            pl.BlockSpec((_BR, do), lambda i: (i, 0)),
            pl.BlockSpec((_BR, 1), lambda i: (i, 0)),
            _full((1, do)),
            _full((1, do)),
            _full((1, do)),
            _full((1, do)),
            _full((do, d1)),
            _full((1, d1)),
            _full((1, d1)),
            _full((1, d1)),
            _full((d1, d2)),
            _full((1, d2)),
            _full((1, d2)),
            _full((1, d2)),
            _full((d2, d3)),
            _full((1, d3)),
            _full((d3, 1)),
            _full((1, 1)),
        ],
        out_specs=pl.BlockSpec((_BR, 1), lambda i: (i, 0)),
        out_shape=jax.ShapeDtypeStruct((N, 1), _F32),
    )(parts, zt, dinv, scale, shift, lng, lnb,
      w1, b1, l1g, l1b, w2, b2, l2g, l2b, w3, b3, w4, b4)


def _bn_fold(bn, bias):
    s = bn["g"] * lax.rsqrt(bn["v"] + 1e-5)
    sh = (bias - bn["m"]) * s + bn["b"]
    return s[None, :], sh[None, :]


def _r2(v):
    return v[None, :]


def kernel(x, edge_index, params):
    p = params
    N, _ = x.shape
    E = edge_index.shape[1]
    row = edge_index[0]
    col = edge_index[1]
    EPT = E // (_NC * _NS)
    row_e = row.reshape(_NC * _NS, EPT // _K, _K)
    col_e = col.reshape(_NC * _NS, EPT // _K, _K)

    deg_parts = _deg_kernel(N, E)(col.reshape(_NC * _NS, EPT))
    dinv = _dinv_tc(deg_parts)

    layers = p["layers"]
    s_in, sh_in = _bn_fold(p["in_bn"], p["in_b"])
    zta, ztb = _k_in(x, p["in_W"], s_in, sh_in, dinv, layers[0]["W"])

    def agg(z):
        do = z.shape[1]
        return _agg_edge_split(N, E, do)(
            z, row_e, col_e, jnp.zeros((25 if do >= 128 else 125, do), _F32))

    s_0, sh_0 = _bn_fold(layers[0]["bn"], layers[0]["b"])
    zt = _k_mid0(agg(zta), agg(ztb), zta, ztb, dinv, s_0, sh_0,
                 _r2(layers[0]["ln"]["g"]), _r2(layers[0]["ln"]["b"]),
                 layers[1]["W"])

    for i, layer in enumerate(layers[1:], start=1):
        parts = agg(zt)
        s_i, sh_i = _bn_fold(layer["bn"], layer["b"])
        lng, lnb = _r2(layer["ln"]["g"]), _r2(layer["ln"]["b"])
        if i + 1 < len(layers):
            zt = _k_mid(parts, zt, dinv, s_i, sh_i, lng, lnb,
                        layers[i + 1]["W"])
        else:
            out = _k_last(
                parts, zt, dinv, s_i, sh_i, lng, lnb,
                p["p_W1"], _r2(p["p_b1"]), _r2(p["p_ln1g"]), _r2(p["p_ln1b"]),
                p["p_W2"], _r2(p["p_b2"]), _r2(p["p_ln2g"]), _r2(p["p_ln2b"]),
                p["p_W3"], _r2(p["p_b3"]), p["p_W4"], _r2(p["p_b4"]))
    return jnp.squeeze(out, -1)


# speedup vs baseline: 25.5209x; 1.1841x over previous
import functools

import jax
import jax.numpy as jnp
from jax import lax
from jax.experimental import pallas as pl
from jax.experimental.pallas import tpu as pltpu
from jax.experimental.pallas import tpu_sc as plsc

_F32 = jnp.float32
_NC = 2
_NS = 16
_K = 80
_BR = 1000


def _sc_mesh():
    return plsc.VectorSubcoreMesh(core_axis_name="c", subcore_axis_name="s",
                                  num_cores=_NC, num_subcores=_NS)


@functools.lru_cache(maxsize=None)
def _deg_kernel(N, E):
    EPT = E // (_NC * _NS)

    def body(col_hbm, out_hbm, colbuf, tacc):
        c = lax.axis_index("c")
        s = lax.axis_index("s")
        t = c * _NS + s
        pltpu.sync_copy(col_hbm.at[t], colbuf)
        zeros16 = jnp.zeros((16,), _F32)

        def zero(i, carry):
            tacc[pl.ds(i * 16, 16)] = zeros16
            return carry

        lax.fori_loop(0, N // 16, zero, 0)
        ones16 = jnp.full((16,), 1.0, _F32)

        def add(i, carry):
            idxv = colbuf[pl.ds(i * 16, 16)]
            plsc.addupdate_scatter(tacc, [idxv], ones16)
            return carry

        lax.fori_loop(0, EPT // 16, add, 0)
        pltpu.sync_copy(tacc, out_hbm.at[t])

    return pl.kernel(
        body,
        out_type=jax.ShapeDtypeStruct((_NC * _NS, N), _F32),
        mesh=_sc_mesh(),
        scratch_types=[
            pltpu.VMEM((EPT,), jnp.int32),
            pltpu.VMEM((N,), _F32),
        ],
        compiler_params=pltpu.CompilerParams(
            use_tc_tiling_on_sc=False, needs_layout_passes=False),
    )


def _agg_pass(zt_hbm, zeros_hbm, out_hbm, idxr, idxc, gbufs, acc,
              gsems, ssems, s, c, nch, RPT, nbuf):
    pltpu.sync_copy(zeros_hbm, acc.at[pl.ds(s * RPT, RPT)])
    plsc.subcore_barrier()

    def gather(j, b):
        pltpu.async_copy(zt_hbm.at[idxr.at[j]], gbufs[b], gsems[b])

    def scatter(j, b):
        pltpu.async_copy(gbufs[b], acc.at[idxc.at[j]], ssems[b], add=True)

    def wait_g(b):
        pltpu.make_async_copy(
            zt_hbm.at[idxr.at[0]], gbufs[b], gsems[b]).wait()

    def wait_s(b):
        pltpu.make_async_copy(
            gbufs[b], acc.at[idxc.at[0]], ssems[b]).wait()

    nrounds, rem = divmod(nch, nbuf)
    for b in range(nbuf):
        gather(b, b)

    def round_body(r, carry):
        j0 = r * nbuf
        for b in range(nbuf):
            wait_g(b)
            scatter(j0 + b, b)
        for b in range(nbuf):
            wait_s(b)
            nj = j0 + nbuf + b

            @pl.when(nj < nch)
            def _():
                gather(nj, b)
        return carry

    lax.fori_loop(0, nrounds, round_body, 0)
    for i in range(rem):
        wait_g(i)
        scatter(nrounds * nbuf + i, i)
    for i in range(rem):
        wait_s(i)
    plsc.subcore_barrier()
    pltpu.sync_copy(acc.at[pl.ds(s * RPT, RPT)],
                    out_hbm.at[c, pl.ds(s * RPT, RPT)])


def _agg_scratch(N, nch, do, nbuf):
    return (
        [pltpu.VMEM((nch, _K), jnp.int32)] * 2
        + [pltpu.VMEM((_K, do), _F32)] * nbuf
        + [pltpu.VMEM_SHARED((N, do), _F32)]
        + [pltpu.SemaphoreType.DMA] * (2 * nbuf)
    )


def _agg_nbuf(do):
    return 3 if do >= 128 else 8


@functools.lru_cache(maxsize=None)
def _agg_edge_split(N, E, do):
    EPT = E // (_NC * _NS)
    nch = EPT // _K
    RPT = N // _NS
    nbuf = _agg_nbuf(do)

    def run(zt, rw, cl, zs, out, idxr, idxc, gbufs, acc, gsems, ssems):
        c = lax.axis_index("c")
        s = lax.axis_index("s")
        t = c * _NS + s
        pltpu.sync_copy(rw.at[t], idxr)
        pltpu.sync_copy(cl.at[t], idxc)
        _agg_pass(zt, zs, out, idxr, idxc, gbufs, acc, gsems, ssems,
                  s, c, nch, RPT, nbuf)

    if nbuf == 3:
        def body(zt, rw, cl, zs, out, idxr, idxc, g0, g1, g2, acc,
                 gs0, gs1, gs2, ss0, ss1, ss2):
            run(zt, rw, cl, zs, out, idxr, idxc, (g0, g1, g2), acc,
                (gs0, gs1, gs2), (ss0, ss1, ss2))
    else:
        def body(zt, rw, cl, zs, out, idxr, idxc,
                 g0, g1, g2, g3, g4, g5, g6, g7, acc,
                 gs0, gs1, gs2, gs3, gs4, gs5, gs6, gs7,
                 ss0, ss1, ss2, ss3, ss4, ss5, ss6, ss7):
            run(zt, rw, cl, zs, out, idxr, idxc,
                (g0, g1, g2, g3, g4, g5, g6, g7), acc,
                (gs0, gs1, gs2, gs3, gs4, gs5, gs6, gs7),
                (ss0, ss1, ss2, ss3, ss4, ss5, ss6, ss7))

    return pl.kernel(
        body,
        out_type=jax.ShapeDtypeStruct((_NC, N, do), _F32),
        mesh=_sc_mesh(),
        scratch_types=_agg_scratch(N, nch, do, nbuf),
        compiler_params=pltpu.CompilerParams(use_tc_tiling_on_sc=False),
    )


@functools.lru_cache(maxsize=None)
def _agg_edge_split2(N, E, do):
    EPT = E // (_NC * _NS)
    nch = EPT // _K
    RPT = N // _NS
    nbuf = _agg_nbuf(do)

    def body(zta, ztb, rw, cl, zs, outa, outb, idxr, idxc, g0, g1, g2, acc,
             gs0, gs1, gs2, ss0, ss1, ss2):
        c = lax.axis_index("c")
        s = lax.axis_index("s")
        t = c * _NS + s
        pltpu.sync_copy(rw.at[t], idxr)
        pltpu.sync_copy(cl.at[t], idxc)
        for zt, out in ((zta, outa), (ztb, outb)):
            _agg_pass(zt, zs, out, idxr, idxc, (g0, g1, g2), acc,
                      (gs0, gs1, gs2), (ss0, ss1, ss2),
                      s, c, nch, RPT, nbuf)

    shape = jax.ShapeDtypeStruct((_NC, N, do), _F32)
    return pl.kernel(
        body,
        out_type=[shape, shape],
        mesh=_sc_mesh(),
        scratch_types=_agg_scratch(N, nch, do, nbuf),
        compiler_params=pltpu.CompilerParams(use_tc_tiling_on_sc=False),
    )


_SQRT1_2 = 0.7071067811865476


def _gelu(x):
    return x * 0.5 * (1.0 + lax.erf(x * _SQRT1_2))


def _full(shape):
    return pl.BlockSpec(shape, lambda i: tuple(0 for _ in shape))


def _dinv_tc(deg_parts):
    N = deg_parts.shape[1]

    def body(d_ref, o_ref):
        d = jnp.sum(d_ref[...], axis=0) + 1.0
        o_ref[...] = lax.rsqrt(d)[:, None]

    return pl.pallas_call(
        body, out_shape=jax.ShapeDtypeStruct((N, 1), _F32))(deg_parts)


def _k_in(x, win, scale, shift, dinv, w0):
    N, DIN = x.shape
    H = win.shape[1]
    HH = H // 2

    def body(x_ref, w_ref, s_ref, sh_ref, dv_ref, w0_ref, oa_ref, ob_ref):
        u = jnp.dot(x_ref[...], w_ref[...], preferred_element_type=_F32)
        u = _gelu(u * s_ref[...] + sh_ref[...])
        zt = dv_ref[...] * jnp.dot(u, w0_ref[...], preferred_element_type=_F32)
        oa_ref[...] = zt[:, :HH]
        ob_ref[...] = zt[:, HH:]

    return pl.pallas_call(
        body,
        grid=(N // _BR,),
        in_specs=[
            pl.BlockSpec((_BR, DIN), lambda i: (i, 0)),
            _full((DIN, H)),
            _full((1, H)),
            _full((1, H)),
            pl.BlockSpec((_BR, 1), lambda i: (i, 0)),
            _full((H, H)),
        ],
        out_specs=[pl.BlockSpec((_BR, HH), lambda i: (i, 0)),
                   pl.BlockSpec((_BR, HH), lambda i: (i, 0))],
        out_shape=[jax.ShapeDtypeStruct((N, HH), _F32),
                   jax.ShapeDtypeStruct((N, HH), _F32)],
    )(x, win, scale, shift, dinv, w0)


def _k_mid0(partsa, partsb, zta, ztb, dinv, scale, shift, lng, lnb, wnext):
    _, N, dh = partsa.shape
    dn = wnext.shape[1]
    do = 2 * dh

    def body(pa_ref, pb_ref, za_ref, zb_ref, dv_ref, s_ref, sh_ref,
             g_ref, b_ref, w_ref, o_ref):
        agg = jnp.concatenate(
            [pa_ref[0] + pa_ref[1] + za_ref[...],
             pb_ref[0] + pb_ref[1] + zb_ref[...]], axis=-1)
        z = dv_ref[...] * agg
        z = _gelu(z * s_ref[...] + sh_ref[...])
        m = jnp.mean(z, axis=-1, keepdims=True)
        zc = z - m
        v = jnp.mean(zc * zc, axis=-1, keepdims=True)
        z = zc * lax.rsqrt(v + 1e-5) * g_ref[...] + b_ref[...]
        o_ref[...] = dv_ref[...] * jnp.dot(
            z, w_ref[...], preferred_element_type=_F32)

    return pl.pallas_call(
        body,
        grid=(N // _BR,),
        in_specs=[
            pl.BlockSpec((2, _BR, dh), lambda i: (0, i, 0)),
            pl.BlockSpec((2, _BR, dh), lambda i: (0, i, 0)),
            pl.BlockSpec((_BR, dh), lambda i: (i, 0)),
            pl.BlockSpec((_BR, dh), lambda i: (i, 0)),
            pl.BlockSpec((_BR, 1), lambda i: (i, 0)),
            _full((1, do)),
            _full((1, do)),
            _full((1, do)),
            _full((1, do)),
            _full((do, dn)),
        ],
        out_specs=pl.BlockSpec((_BR, dn), lambda i: (i, 0)),
        out_shape=jax.ShapeDtypeStruct((N, dn), _F32),
    )(partsa, partsb, zta, ztb, dinv, scale, shift, lng, lnb, wnext)


def _k_mid(parts, zt, dinv, scale, shift, lng, lnb, wnext):
    _, N, do = parts.shape
    dn = wnext.shape[1]

    def body(p_ref, zt_ref, dv_ref, s_ref, sh_ref, g_ref, b_ref, w_ref, o_ref):
        agg = p_ref[0] + p_ref[1]
        z = dv_ref[...] * (agg + zt_ref[...])
        z = _gelu(z * s_ref[...] + sh_ref[...])
        m = jnp.mean(z, axis=-1, keepdims=True)
        zc = z - m
        v = jnp.mean(zc * zc, axis=-1, keepdims=True)
        z = zc * lax.rsqrt(v + 1e-5) * g_ref[...] + b_ref[...]
        o_ref[...] = dv_ref[...] * jnp.dot(
            z, w_ref[...], preferred_element_type=_F32)

    return pl.pallas_call(
        body,
        grid=(N // _BR,),
        in_specs=[
            pl.BlockSpec((2, _BR, do), lambda i: (0, i, 0)),
            pl.BlockSpec((_BR, do), lambda i: (i, 0)),
            pl.BlockSpec((_BR, 1), lambda i: (i, 0)),
            _full((1, do)),
            _full((1, do)),
            _full((1, do)),
            _full((1, do)),
            _full((do, dn)),
        ],
        out_specs=pl.BlockSpec((_BR, dn), lambda i: (i, 0)),
        out_shape=jax.ShapeDtypeStruct((N, dn), _F32),
    )(parts, zt, dinv, scale, shift, lng, lnb, wnext)


def _k_last(parts, zt, dinv, scale, shift, lng, lnb,
            w1, b1, l1g, l1b, w2, b2, l2g, l2b, w3, b3, w4, b4):
    _, N, do = parts.shape
    d1 = w1.shape[1]
    d2 = w2.shape[1]
    d3 = w3.shape[1]

    def _ln(z, g, b):
        m = jnp.mean(z, axis=-1, keepdims=True)
        zc = z - m
        v = jnp.mean(zc * zc, axis=-1, keepdims=True)
        return zc * lax.rsqrt(v + 1e-5) * g + b

    def body(p_ref, zt_ref, dv_ref, s_ref, sh_ref, g_ref, b_ref,
             w1_ref, b1_ref, l1g_ref, l1b_ref, w2_ref, b2_ref,
             l2g_ref, l2b_ref, w3_ref, b3_ref, w4_ref, b4_ref, o_ref):
        z = dv_ref[...] * (p_ref[0] + p_ref[1] + zt_ref[...])
        z = _gelu(z * s_ref[...] + sh_ref[...])
        h = _ln(z, g_ref[...], b_ref[...])
        q = jnp.dot(h, w1_ref[...], preferred_element_type=_F32) + b1_ref[...]
        q = _gelu(_ln(q, l1g_ref[...], l1b_ref[...]))
        q = jnp.dot(q, w2_ref[...], preferred_element_type=_F32) + b2_ref[...]
        q = _gelu(_ln(q, l2g_ref[...], l2b_ref[...]))
        q = _gelu(jnp.dot(q, w3_ref[...], preferred_element_type=_F32)
                  + b3_ref[...])
        o_ref[...] = jnp.dot(q, w4_ref[...],
                             preferred_element_type=_F32) + b4_ref[...]

    return pl.pallas_call(
        body,
        grid=(N // _BR,),
        in_specs=[
            pl.BlockSpec((2, _BR, do), lambda i: (0, i, 0)),
            pl.BlockSpec((_BR, do), lambda i: (i, 0)),
            pl.BlockSpec((_BR, 1), lambda i: (i, 0)),
            _full((1, do)),
            _full((1, do)),
            _full((1, do)),
            _full((1, do)),
            _full((do, d1)),
            _full((1, d1)),
            _full((1, d1)),
            _full((1, d1)),
            _full((d1, d2)),
            _full((1, d2)),
            _full((1, d2)),
            _full((1, d2)),
            _full((d2, d3)),
            _full((1, d3)),
            _full((d3, 1)),
            _full((1, 1)),
        ],
        out_specs=pl.BlockSpec((_BR, 1), lambda i: (i, 0)),
        out_shape=jax.ShapeDtypeStruct((N, 1), _F32),
    )(parts, zt, dinv, scale, shift, lng, lnb,
      w1, b1, l1g, l1b, w2, b2, l2g, l2b, w3, b3, w4, b4)


def _bn_fold(bn, bias):
    s = bn["g"] * lax.rsqrt(bn["v"] + 1e-5)
    sh = (bias - bn["m"]) * s + bn["b"]
    return s[None, :], sh[None, :]


def _r2(v):
    return v[None, :]


def kernel(x, edge_index, params):
    p = params
    N, _ = x.shape
    E = edge_index.shape[1]
    row = edge_index[0]
    col = edge_index[1]
    EPT = E // (_NC * _NS)
    row_e = row.reshape(_NC * _NS, EPT // _K, _K)
    col_e = col.reshape(_NC * _NS, EPT // _K, _K)

    deg_parts = _deg_kernel(N, E)(col.reshape(_NC * _NS, EPT))
    dinv = _dinv_tc(deg_parts)

    layers = p["layers"]
    s_in, sh_in = _bn_fold(p["in_bn"], p["in_b"])
    zta, ztb = _k_in(x, p["in_W"], s_in, sh_in, dinv, layers[0]["W"])

    RPT = N // _NS

    def agg(z):
        do = z.shape[1]
        return _agg_edge_split(N, E, do)(
            z, row_e, col_e, jnp.zeros((RPT, do), _F32))

    s_0, sh_0 = _bn_fold(layers[0]["bn"], layers[0]["b"])
    partsa, partsb = _agg_edge_split2(N, E, zta.shape[1])(
        zta, ztb, row_e, col_e, jnp.zeros((RPT, zta.shape[1]), _F32))
    zt = _k_mid0(partsa, partsb, zta, ztb, dinv, s_0, sh_0,
                 _r2(layers[0]["ln"]["g"]), _r2(layers[0]["ln"]["b"]),
                 layers[1]["W"])

    for i, layer in enumerate(layers[1:], start=1):
        parts = agg(zt)
        s_i, sh_i = _bn_fold(layer["bn"], layer["b"])
        lng, lnb = _r2(layer["ln"]["g"]), _r2(layer["ln"]["b"])
        if i + 1 < len(layers):
            zt = _k_mid(parts, zt, dinv, s_i, sh_i, lng, lnb,
                        layers[i + 1]["W"])
        else:
            out = _k_last(
                parts, zt, dinv, s_i, sh_i, lng, lnb,
                p["p_W1"], _r2(p["p_b1"]), _r2(p["p_ln1g"]), _r2(p["p_ln1b"]),
                p["p_W2"], _r2(p["p_b2"]), _r2(p["p_ln2g"]), _r2(p["p_ln2b"]),
                p["p_W3"], _r2(p["p_b3"]), p["p_W4"], _r2(p["p_b4"]))
    return jnp.squeeze(out, -1)
